# baseline (speedup 1.0000x reference)
import functools

import jax
import jax.numpy as jnp
from jax import lax
from jax.experimental import pallas as pl
from jax.experimental.pallas import tpu as pltpu
from jax.experimental.pallas import tpu_sc as plsc

NUM_EMBEDDINGS = 100000
EMBED_DIM = 128
BATCH = 4096
HIST = 200

NC = 2
NS = 16
NW = NC * NS

CHUNK = 128
N_FLAT = BATCH * HIST
ROWS_PER_W = N_FLAT // NW
CHUNKS_PER_W = ROWS_PER_W // CHUNK
NBUF = 5
LOOKAHEAD = 3
DRAINLAG = NBUF - LOOKAHEAD
NGROUPS = CHUNKS_PER_W // NBUF


def _gather_body(x_hbm, table_hbm, out_hbm, idx_v, rows_v, gsems, ssems):
    wid = lax.axis_index("s") * NC + lax.axis_index("c")
    base_chunk = wid * CHUNKS_PER_W
    pltpu.sync_copy(x_hbm.at[pl.ds(base_chunk, CHUNKS_PER_W)], idx_v)

    def out_slice(j):
        return out_hbm.at[pl.ds((base_chunk + j) * CHUNK, CHUNK)]

    def fire_gather(j, b):
        pltpu.async_copy(table_hbm.at[idx_v.at[j]], rows_v.at[b], gsems.at[b])

    def wait_gather(j, b):
        pltpu.make_async_copy(
            table_hbm.at[idx_v.at[j]], rows_v.at[b], gsems.at[b]
        ).wait()

    def fire_store(j, b):
        pltpu.async_copy(rows_v.at[b], out_slice(j), ssems.at[b])

    def wait_store(j, b):
        pltpu.make_async_copy(rows_v.at[b], out_slice(j), ssems.at[b]).wait()

    def step(j, b, do_drain, do_fire):
        if do_drain:
            wait_store(j - DRAINLAG, (j - DRAINLAG) % NBUF)
        if do_fire:
            fire_gather(j + LOOKAHEAD, (j + LOOKAHEAD) % NBUF)
        wait_gather(j, b)
        fire_store(j, b)

    for b in range(LOOKAHEAD):
        fire_gather(b, b)
    for b in range(NBUF):
        step(b, b, do_drain=b >= DRAINLAG, do_fire=True)

    def group(g, carry):
        for b in range(NBUF):
            j = g * NBUF + b
            step(j, b, do_drain=True, do_fire=True)
        return carry

    lax.fori_loop(1, NGROUPS - 1, group, 0, unroll=False)

    for b in range(NBUF):
        j = (NGROUPS - 1) * NBUF + b
        step(j, b, do_drain=True, do_fire=(j + LOOKAHEAD < CHUNKS_PER_W))
    for j in range(CHUNKS_PER_W - DRAINLAG, CHUNKS_PER_W):
        wait_store(j, j % NBUF)


@jax.jit
def _embed_lookup(x2d, table):
    mesh = plsc.VectorSubcoreMesh(
        core_axis_name="c", subcore_axis_name="s", num_cores=NC, num_subcores=NS
    )
    run = pl.kernel(
        _gather_body,
        out_type=jax.ShapeDtypeStruct((N_FLAT, EMBED_DIM), jnp.float32),
        mesh=mesh,
        scratch_types=[
            pltpu.VMEM((CHUNKS_PER_W, CHUNK), jnp.int32),
            pltpu.VMEM((NBUF, CHUNK, EMBED_DIM), jnp.float32),
            pltpu.SemaphoreType.DMA((NBUF,)),
            pltpu.SemaphoreType.DMA((NBUF,)),
        ],
    )
    return run(x2d, table)


def kernel(x, table):
    x2d = x.reshape(N_FLAT // CHUNK, CHUNK)
    out = _embed_lookup(x2d, table)
    return out.reshape(BATCH, HIST, EMBED_DIM)


# speedup vs baseline: 1.0001x; 1.0001x over previous
import jax
import jax.numpy as jnp
from jax import lax
from jax.experimental import pallas as pl
from jax.experimental.pallas import tpu as pltpu
from jax.experimental.pallas import tpu_sc as plsc

NUM_EMBEDDINGS = 100000
EMBED_DIM = 128
BATCH = 4096
HIST = 200

NC = 2
NS = 16
NW = NC * NS

STEP_ROWS = 256
N_FLAT = BATCH * HIST
ROWS_PER_W = N_FLAT // NW
STEPS = ROWS_PER_W // STEP_ROWS
P = 3
LOOKAHEAD = 2
DRAINLAG = P - LOOKAHEAD


def _gather_body(x_hbm, table_hbm, out_hbm, idx_v, rows_v, gsems, ssems):
    wid = lax.axis_index("s") * NC + lax.axis_index("c")
    base_row = wid * ROWS_PER_W
    pltpu.sync_copy(x_hbm.at[pl.ds(base_row, ROWS_PER_W)], idx_v)

    def gather_args(t, p):
        return (
            table_hbm.at[idx_v.at[pl.ds(STEP_ROWS * t, STEP_ROWS)]],
            rows_v.at[p],
            gsems.at[p],
        )

    def store_args(t, p):
        return (
            rows_v.at[p],
            out_hbm.at[pl.ds(base_row + STEP_ROWS * t, STEP_ROWS)],
            ssems.at[p],
        )

    def step(t, b, do_drain, do_fire):
        if do_drain:
            pltpu.make_async_copy(
                *store_args(t - DRAINLAG, (t - DRAINLAG) % P)
            ).wait()
        if do_fire:
            pltpu.async_copy(*gather_args(t + LOOKAHEAD, (t + LOOKAHEAD) % P))
        pltpu.make_async_copy(*gather_args(t, b)).wait()
        pltpu.async_copy(*store_args(t, b))

    for t in range(LOOKAHEAD):
        pltpu.async_copy(*gather_args(t, t % P))
    step(0, 0, do_drain=False, do_fire=True)

    def group(g, carry):
        for r in range(1, P + 1):
            t = P * g + r
            step(t, r % P, do_drain=True, do_fire=True)
        return carry

    lax.fori_loop(0, (STEPS - 4) // P, group, 0, unroll=False)

    for t in range(STEPS - 3, STEPS):
        step(t, t % P, do_drain=True, do_fire=(t + LOOKAHEAD < STEPS))
    for t in range(STEPS - DRAINLAG, STEPS):
        pltpu.make_async_copy(*store_args(t, t % P)).wait()


@jax.jit
def _embed_lookup(x1d, table):
    mesh = plsc.VectorSubcoreMesh(
        core_axis_name="c", subcore_axis_name="s", num_cores=NC, num_subcores=NS
    )
    run = pl.kernel(
        _gather_body,
        out_type=jax.ShapeDtypeStruct((N_FLAT, EMBED_DIM), jnp.float32),
        mesh=mesh,
        scratch_types=[
            pltpu.VMEM((ROWS_PER_W,), jnp.int32),
            pltpu.VMEM((P, STEP_ROWS, EMBED_DIM), jnp.float32),
            pltpu.SemaphoreType.DMA((P,)),
            pltpu.SemaphoreType.DMA((P,)),
        ],
    )
    return run(x1d, table)


def kernel(x, table):
    x1d = x.reshape(N_FLAT)
    out = _embed_lookup(x1d, table)
    return out.reshape(BATCH, HIST, EMBED_DIM)
